# baseline (speedup 1.0000x reference)
import functools

import jax
import jax.numpy as jnp
from jax import lax
from jax.experimental import pallas as pl
from jax.experimental.pallas import tpu as pltpu
from jax.experimental.pallas import tpu_sc as plsc

NC, NS = 2, 16
NW = NC * NS
CH = 128
HP = lax.Precision.HIGHEST


def _make_segsum(n_pad, e_pad, D, with_counts=False, n_tables=1, ch=CH):
    G = 4
    per_w = e_pad // NW
    n_chunks = per_w // ch
    n_groups = n_chunks // G
    assert n_chunks % (2 * G) == 0 and n_groups % 2 == 0
    rpt = n_pad // NS
    mesh = plsc.VectorSubcoreMesh(
        core_axis_name="c", subcore_axis_name="s", num_cores=NC, num_subcores=NS
    )
    out_type = [jax.ShapeDtypeStruct((NC, n_pad, D), jnp.float32)
                for _ in range(n_tables)]
    scratch = [
        pltpu.VMEM((2 * G, ch), jnp.int32),
        pltpu.VMEM((2 * G, ch), jnp.int32),
        pltpu.VMEM((2 * ch, D), jnp.float32),
        pltpu.VMEM_SHARED((n_pad, D), jnp.float32),
        pltpu.SemaphoreType.DMA,
        pltpu.SemaphoreType.DMA,
        pltpu.SemaphoreType.DMA,
        pltpu.SemaphoreType.DMA,
        pltpu.SemaphoreType.DMA,
    ]
    if with_counts:
        out_type.append(jax.ShapeDtypeStruct((NW, n_pad), jnp.float32))
        scratch.append(pltpu.VMEM((n_pad,), jnp.float32))

    @functools.partial(
        pl.kernel, out_type=tuple(out_type), mesh=mesh, scratch_types=scratch,
        compiler_params=pltpu.CompilerParams(needs_layout_passes=False),
    )
    def seg(*args):
        tables = args[:n_tables]
        srcs, dsts, zeros = args[n_tables:n_tables + 3]
        outs = args[n_tables + 3:2 * n_tables + 3]
        rest = args[2 * n_tables + 3:]
        if with_counts:
            cnt_out, *rest = rest
            *rest, cnt_l = rest
        s_l, d_l, rows, acc, isem = rest[:5]
        gsem = rest[5:7]
        ssem = rest[7:9]
        bufs = [rows.at[pl.ds(b * ch, ch)] for b in range(2)]
        c = lax.axis_index("c")
        s = lax.axis_index("s")
        w = s * NC + c
        row0 = w * n_chunks

        def load_group(g, slot):
            pltpu.async_copy(srcs.at[pl.ds(row0 + g * G, G)],
                             s_l.at[pl.ds(slot * G, G)], isem)
            pltpu.async_copy(dsts.at[pl.ds(row0 + g * G, G)],
                             d_l.at[pl.ds(slot * G, G)], isem)

        def wait_group(g, slot):
            pltpu.make_async_copy(srcs.at[pl.ds(row0 + g * G, G)],
                                  s_l.at[pl.ds(slot * G, G)], isem).wait()
            pltpu.make_async_copy(dsts.at[pl.ds(row0 + g * G, G)],
                                  d_l.at[pl.ds(slot * G, G)], isem).wait()

        if with_counts:
            def zbody(j, carry):
                cnt_l[pl.ds(j * 16, 16)] = jnp.zeros((16,), jnp.float32)
                return carry
            lax.fori_loop(0, n_pad // 16, zbody, 0)

        for t in range(n_tables):
            table, out = tables[t], outs[t]
            pltpu.sync_copy(zeros, acc.at[pl.ds(s * rpt, rpt)])
            plsc.subcore_barrier()

            load_group(0, 0)
            wait_group(0, 0)
            pltpu.async_copy(table.at[s_l.at[0]], bufs[0], gsem[0])
            pltpu.async_copy(table.at[s_l.at[1]], bufs[1], gsem[1])

            def body(gp, carry):
                for p in range(2):
                    g = gp * 2 + p
                    for k in range(G):
                        b = k % 2
                        r = p * G + k
                        nr = r + 1 if k < G - 1 else (1 - p) * G
                        if k == 0:
                            @pl.when(g + 1 < n_groups)
                            def _():
                                load_group(g + 1, 1 - p)
                        pltpu.make_async_copy(
                            table.at[s_l.at[r]], bufs[b], gsem[b]).wait()
                        if k == 0:
                            @pl.when(g >= 1)
                            def _():
                                pltpu.async_copy(
                                    table.at[s_l.at[nr]], bufs[1 - b],
                                    gsem[1 - b])
                        elif k < G - 1:
                            pltpu.async_copy(
                                table.at[s_l.at[nr]], bufs[1 - b], gsem[1 - b])
                        else:
                            @pl.when(g + 1 < n_groups)
                            def _():
                                wait_group(g + 1, 1 - p)
                                pltpu.async_copy(
                                    table.at[s_l.at[nr]], bufs[1 - b],
                                    gsem[1 - b])
                        pltpu.sync_copy(bufs[b], acc.at[d_l.at[r]], add=True)

                        if with_counts and t == 0:
                            one = jnp.full((16,), 1.0, jnp.float32)
                            for j in range(ch // 16):
                                plsc.addupdate_scatter(
                                    cnt_l, [d_l[r, pl.ds(j * 16, 16)]], one)
                return carry

            lax.fori_loop(0, n_groups // 2, body, 0)
            plsc.subcore_barrier()
            pltpu.sync_copy(
                acc.at[pl.ds(s * rpt, rpt)], out.at[c].at[pl.ds(s * rpt, rpt)]
            )
        if with_counts:
            pltpu.sync_copy(cnt_l, cnt_out.at[w])

    return seg


def _tc_call(body, grid, in_specs, out_specs, out_shapes):
    return pl.pallas_call(
        body, grid=grid, in_specs=in_specs, out_specs=out_specs,
        out_shape=out_shapes,
    )


def _full(shape):
    return pl.BlockSpec(shape, lambda i: (0,) * len(shape))


def _rows(m, d):
    return pl.BlockSpec((m, d), lambda i: (i, 0))


def _agg_spec(m, d, n_blocks=NC):
    return pl.BlockSpec((n_blocks, m, d), lambda i: (0, i, 0))


def _stats_body_init(i, refs):
    @pl.when(i == 0)
    def _():
        for r in refs:
            r[...] = jnp.zeros_like(r)


def _bn_relu(o, sum_ref, sq_ref, gamma_ref, beta_ref, n):
    mu = sum_ref[...] * (1.0 / n)
    var = sq_ref[...] * (1.0 / n) - mu * mu
    scale = gamma_ref[...] * lax.rsqrt(var + 1e-5)
    return jnp.maximum((o - mu) * scale + beta_ref[...], 0.0)


def kernel(x, edge_index, W_l1, b_l1, W_r1, gamma1, beta1,
           W_l2, b_l2, W_r2, gamma2, beta2, W_l3, b_l3, W_r3, gamma3, beta3):
    N = x.shape[0]
    E = edge_index.shape[1]
    M = 1000
    R = N // M
    n_pad = ((N + 2047) // 2048) * 2048
    e_pad = ((E + NW * CH * 16 - 1) // (NW * CH * 16)) * (NW * CH * 16)

    CH2 = 128
    assert e_pad % (NW * CH2 * 16) == 0
    ei = edge_index.astype(jnp.int32)
    src_f = jnp.concatenate([ei[0], jnp.zeros((e_pad - E,), jnp.int32)])
    dst_f = jnp.concatenate([ei[1], jnp.full((e_pad - E,), N, jnp.int32)])
    src, dst = src_f.reshape(-1, CH), dst_f.reshape(-1, CH)
    src2, dst2 = src_f.reshape(-1, CH2), dst_f.reshape(-1, CH2)

    z128 = jnp.zeros((n_pad // NS, 128), jnp.float32)
    seg_cnt = _make_segsum(n_pad, e_pad, 128, with_counts=True)
    seg_two = _make_segsum(n_pad, e_pad, 128, n_tables=2)
    seg_one = _make_segsum(n_pad, e_pad, 128)

    agg1, cnt_p = seg_cnt(x, src, dst, z128)
    cnt = cnt_p.sum(axis=0).reshape(n_pad, 1)[:N]

    def tc1_l1(agg_ref, cnt_ref, x_ref, wl_ref, b_ref, wr_ref,
               out_ref, sum_ref, sq_ref, cinv_ref):
        i = pl.program_id(0)
        agg = agg_ref[0] + agg_ref[1]
        cinv = 1.0 / jnp.maximum(cnt_ref[...], 1.0)
        mean = agg * cinv
        o = (jnp.dot(mean, wl_ref[...], precision=HP, preferred_element_type=jnp.float32)
             + b_ref[...]
             + jnp.dot(x_ref[...], wr_ref[...], precision=HP, preferred_element_type=jnp.float32))
        out_ref[...] = o
        cinv_ref[...] = cinv
        _stats_body_init(i, (sum_ref, sq_ref))
        sum_ref[...] += jnp.sum(o, axis=0, keepdims=True)
        sq_ref[...] += jnp.sum(o * o, axis=0, keepdims=True)

    out1, sum1, sq1, cinv = _tc_call(
        tc1_l1, (R,),
        [_agg_spec(M, 128), _rows(M, 1), _rows(M, 128), _full((128, 512)),
         _full((1, 512)), _full((128, 512))],
        [_rows(M, 512), _full((1, 512)), _full((1, 512)), _rows(M, 1)],
        [jax.ShapeDtypeStruct((N, 512), jnp.float32),
         jax.ShapeDtypeStruct((1, 512), jnp.float32),
         jax.ShapeDtypeStruct((1, 512), jnp.float32),
         jax.ShapeDtypeStruct((N, 1), jnp.float32)],
    )(agg1, cnt, x, W_l1, b_l1.reshape(1, -1), W_r1)

    def tc2_l1(o_ref, sum_ref, sq_ref, g_ref, be_ref, wl_ref, wr_ref,
               pa_ref, pb_ref, r_ref):
        h = _bn_relu(o_ref[...], sum_ref, sq_ref, g_ref, be_ref, float(N))
        p = jnp.dot(h, wl_ref[...], precision=HP, preferred_element_type=jnp.float32)
        pa_ref[...] = p[:, :128]
        pb_ref[...] = p[:, 128:]
        r_ref[...] = jnp.dot(h, wr_ref[...], precision=HP, preferred_element_type=jnp.float32)

    p2a, p2b, r2 = _tc_call(
        tc2_l1, (R,),
        [_rows(M, 512), _full((1, 512)), _full((1, 512)), _full((1, 512)),
         _full((1, 512)), _full((512, 256)), _full((512, 256))],
        [_rows(M, 128), _rows(M, 128), _rows(M, 256)],
        [jax.ShapeDtypeStruct((N, 128), jnp.float32),
         jax.ShapeDtypeStruct((N, 128), jnp.float32),
         jax.ShapeDtypeStruct((N, 256), jnp.float32)],
    )(out1, sum1, sq1, gamma1.reshape(1, -1), beta1.reshape(1, -1), W_l2, W_r2)

    agg2a, agg2b = seg_two(p2a, p2b, src2, dst2, z128)

    def tc1_l2(aa_ref, ab_ref, cinv_ref, r_ref, b_ref, out_ref, sum_ref, sq_ref):
        i = pl.program_id(0)
        cinv = cinv_ref[...]
        a = (aa_ref[0] + aa_ref[1]) * cinv
        b2 = (ab_ref[0] + ab_ref[1]) * cinv
        o = jnp.concatenate([a, b2], axis=1) + b_ref[...] + r_ref[...]
        out_ref[...] = o
        _stats_body_init(i, (sum_ref, sq_ref))
        sum_ref[...] += jnp.sum(o, axis=0, keepdims=True)
        sq_ref[...] += jnp.sum(o * o, axis=0, keepdims=True)

    out2, sum2, sq2 = _tc_call(
        tc1_l2, (R,),
        [_agg_spec(M, 128), _agg_spec(M, 128), _rows(M, 1), _rows(M, 256),
         _full((1, 256))],
        [_rows(M, 256), _full((1, 256)), _full((1, 256))],
        [jax.ShapeDtypeStruct((N, 256), jnp.float32),
         jax.ShapeDtypeStruct((1, 256), jnp.float32),
         jax.ShapeDtypeStruct((1, 256), jnp.float32)],
    )(agg2a, agg2b, cinv, r2, b_l2.reshape(1, -1))

    def tc2_l2(o_ref, sum_ref, sq_ref, g_ref, be_ref, wl_ref, wr_ref,
               p_ref, r_ref):
        h = _bn_relu(o_ref[...], sum_ref, sq_ref, g_ref, be_ref, float(N))
        p_ref[...] = jnp.dot(h, wl_ref[...], precision=HP, preferred_element_type=jnp.float32)
        r_ref[...] = jnp.dot(h, wr_ref[...], precision=HP, preferred_element_type=jnp.float32)

    p3, r3 = _tc_call(
        tc2_l2, (R,),
        [_rows(M, 256), _full((1, 256)), _full((1, 256)), _full((1, 256)),
         _full((1, 256)), _full((256, 128)), _full((256, 128))],
        [_rows(M, 128), _rows(M, 128)],
        [jax.ShapeDtypeStruct((N, 128), jnp.float32),
         jax.ShapeDtypeStruct((N, 128), jnp.float32)],
    )(out2, sum2, sq2, gamma2.reshape(1, -1), beta2.reshape(1, -1), W_l3, W_r3)

    agg3, = seg_one(p3, src2, dst2, z128)

    def tc1_l3(a_ref, cinv_ref, r_ref, b_ref, out_ref, sum_ref, sq_ref):
        i = pl.program_id(0)
        o = (a_ref[0] + a_ref[1]) * cinv_ref[...] + b_ref[...] + r_ref[...]
        out_ref[...] = o
        _stats_body_init(i, (sum_ref, sq_ref))
        sum_ref[...] += jnp.sum(o, axis=0, keepdims=True)
        sq_ref[...] += jnp.sum(o * o, axis=0, keepdims=True)

    out3, sum3, sq3 = _tc_call(
        tc1_l3, (R,),
        [_agg_spec(M, 128), _rows(M, 1), _rows(M, 128), _full((1, 128))],
        [_rows(M, 128), _full((1, 128)), _full((1, 128))],
        [jax.ShapeDtypeStruct((N, 128), jnp.float32),
         jax.ShapeDtypeStruct((1, 128), jnp.float32),
         jax.ShapeDtypeStruct((1, 128), jnp.float32)],
    )(agg3, cinv, r3, b_l3.reshape(1, -1))

    def tc2_l3(o_ref, sum_ref, sq_ref, g_ref, be_ref, h_ref):
        h_ref[...] = _bn_relu(o_ref[...], sum_ref, sq_ref, g_ref, be_ref, float(N))

    h3 = _tc_call(
        tc2_l3, (R,),
        [_rows(M, 128), _full((1, 128)), _full((1, 128)), _full((1, 128)),
         _full((1, 128))],
        [_rows(M, 128)],
        [jax.ShapeDtypeStruct((N, 128), jnp.float32)],
    )(out3, sum3, sq3, gamma3.reshape(1, -1), beta3.reshape(1, -1))[0]

    return h3


# speedup vs baseline: 1.2371x; 1.2371x over previous
import functools

import jax
import jax.numpy as jnp
from jax import lax
from jax.experimental import pallas as pl
from jax.experimental.pallas import tpu as pltpu
from jax.experimental.pallas import tpu_sc as plsc

NC, NS = 2, 16
NW = NC * NS
CH = 128
HP = lax.Precision.HIGHEST


def _make_segsum(n_pad, e_pad, D, with_counts=False):
    per_w = e_pad // NW
    n_chunks = per_w // CH
    rpt = n_pad // NS
    mesh = plsc.VectorSubcoreMesh(
        core_axis_name="c", subcore_axis_name="s", num_cores=NC, num_subcores=NS
    )
    out_type = [jax.ShapeDtypeStruct((NC, n_pad, D), jnp.float32)]
    scratch = [
        pltpu.VMEM((CH,), jnp.int32),
        pltpu.VMEM((CH,), jnp.int32),
        pltpu.VMEM((CH, D), jnp.float32),
        pltpu.VMEM_SHARED((n_pad, D), jnp.float32),
        pltpu.SemaphoreType.DMA,
    ]
    if with_counts:
        out_type.append(jax.ShapeDtypeStruct((NW, n_pad), jnp.float32))
        scratch.append(pltpu.VMEM((n_pad,), jnp.float32))

    @functools.partial(
        pl.kernel, out_type=tuple(out_type) if with_counts else out_type[0],
        mesh=mesh, scratch_types=scratch,
        compiler_params=pltpu.CompilerParams(needs_layout_passes=False),
    )
    def seg(table, srcs, dsts, zeros, out, *rest):
        if with_counts:
            cnt_out, s_v, d_v, rows_v, acc, sem, cnt_l = rest
        else:
            s_v, d_v, rows_v, acc, sem = rest
        c = lax.axis_index("c")
        s = lax.axis_index("s")
        w = s * NC + c
        pltpu.sync_copy(zeros, acc.at[pl.ds(s * rpt, rpt)])
        if with_counts:
            def zbody(j, carry):
                cnt_l[pl.ds(j * 16, 16)] = jnp.zeros((16,), jnp.float32)
                return carry
            lax.fori_loop(0, n_pad // 16, zbody, 0)
        plsc.subcore_barrier()

        def body(i, carry):
            base = w * per_w + i * CH
            pltpu.sync_copy(srcs.at[pl.ds(base, CH)], s_v)
            pltpu.sync_copy(dsts.at[pl.ds(base, CH)], d_v)
            pltpu.async_copy(table.at[s_v], rows_v, sem).wait()
            pltpu.sync_copy(rows_v, acc.at[d_v], add=True)
            if with_counts:
                one = jnp.full((16,), 1.0, jnp.float32)
                for j in range(CH // 16):
                    plsc.addupdate_scatter(cnt_l, [d_v[pl.ds(j * 16, 16)]], one)
            return carry

        lax.fori_loop(0, n_chunks, body, 0)
        plsc.subcore_barrier()
        pltpu.sync_copy(
            acc.at[pl.ds(s * rpt, rpt)], out.at[c].at[pl.ds(s * rpt, rpt)]
        )
        if with_counts:
            pltpu.sync_copy(cnt_l, cnt_out.at[w])

    return seg


def _tc_call(body, grid, in_specs, out_specs, out_shapes):
    return pl.pallas_call(
        body, grid=grid, in_specs=in_specs, out_specs=out_specs,
        out_shape=out_shapes,
    )


def _full(shape):
    return pl.BlockSpec(shape, lambda i: (0,) * len(shape))


def _rows(m, d):
    return pl.BlockSpec((m, d), lambda i: (i, 0))


def _agg_spec(m, d, n_blocks=NC):
    return pl.BlockSpec((n_blocks, m, d), lambda i: (0, i, 0))


def _stats_body_init(i, refs):
    @pl.when(i == 0)
    def _():
        for r in refs:
            r[...] = jnp.zeros_like(r)


def _bn_relu(o, sum_ref, sq_ref, gamma_ref, beta_ref, n):
    mu = sum_ref[...] * (1.0 / n)
    var = sq_ref[...] * (1.0 / n) - mu * mu
    scale = gamma_ref[...] * lax.rsqrt(var + 1e-5)
    return jnp.maximum((o - mu) * scale + beta_ref[...], 0.0)


def kernel(x, edge_index, W_l1, b_l1, W_r1, gamma1, beta1,
           W_l2, b_l2, W_r2, gamma2, beta2, W_l3, b_l3, W_r3, gamma3, beta3):
    N = x.shape[0]
    E = edge_index.shape[1]
    M = 1000
    R = N // M
    n_pad = ((N + 2047) // 2048) * 2048
    e_pad = ((E + NW * CH - 1) // (NW * CH)) * (NW * CH)

    ei = edge_index.astype(jnp.int32)
    src = jnp.concatenate([ei[0], jnp.zeros((e_pad - E,), jnp.int32)])
    dst = jnp.concatenate([ei[1], jnp.full((e_pad - E,), N, jnp.int32)])

    z128 = jnp.zeros((n_pad // NS, 128), jnp.float32)

    seg_cnt = _make_segsum(n_pad, e_pad, 128, with_counts=True)
    seg128 = _make_segsum(n_pad, e_pad, 128)

    agg1, cnt_p = seg_cnt(x, src, dst, z128)
    cnt = cnt_p.sum(axis=0).reshape(n_pad, 1)[:N]

    def tc1_l1(agg_ref, cnt_ref, x_ref, wl_ref, b_ref, wr_ref,
               out_ref, sum_ref, sq_ref, cinv_ref):
        i = pl.program_id(0)
        agg = agg_ref[0] + agg_ref[1]
        cinv = 1.0 / jnp.maximum(cnt_ref[...], 1.0)
        mean = agg * cinv
        o = (jnp.dot(mean, wl_ref[...], precision=HP, preferred_element_type=jnp.float32)
             + b_ref[...]
             + jnp.dot(x_ref[...], wr_ref[...], precision=HP, preferred_element_type=jnp.float32))
        out_ref[...] = o
        cinv_ref[...] = cinv
        _stats_body_init(i, (sum_ref, sq_ref))
        sum_ref[...] += jnp.sum(o, axis=0, keepdims=True)
        sq_ref[...] += jnp.sum(o * o, axis=0, keepdims=True)

    out1, sum1, sq1, cinv = _tc_call(
        tc1_l1, (R,),
        [_agg_spec(M, 128), _rows(M, 1), _rows(M, 128), _full((128, 512)),
         _full((1, 512)), _full((128, 512))],
        [_rows(M, 512), _full((1, 512)), _full((1, 512)), _rows(M, 1)],
        [jax.ShapeDtypeStruct((N, 512), jnp.float32),
         jax.ShapeDtypeStruct((1, 512), jnp.float32),
         jax.ShapeDtypeStruct((1, 512), jnp.float32),
         jax.ShapeDtypeStruct((N, 1), jnp.float32)],
    )(agg1, cnt, x, W_l1, b_l1.reshape(1, -1), W_r1)

    def tc2_l1(o_ref, sum_ref, sq_ref, g_ref, be_ref, wl_ref, wr_ref,
               pa_ref, pb_ref, r_ref):
        h = _bn_relu(o_ref[...], sum_ref, sq_ref, g_ref, be_ref, float(N))
        p = jnp.dot(h, wl_ref[...], precision=HP, preferred_element_type=jnp.float32)
        pa_ref[...] = p[:, :128]
        pb_ref[...] = p[:, 128:]
        r_ref[...] = jnp.dot(h, wr_ref[...], precision=HP, preferred_element_type=jnp.float32)

    p2a, p2b, r2 = _tc_call(
        tc2_l1, (R,),
        [_rows(M, 512), _full((1, 512)), _full((1, 512)), _full((1, 512)),
         _full((1, 512)), _full((512, 256)), _full((512, 256))],
        [_rows(M, 128), _rows(M, 128), _rows(M, 256)],
        [jax.ShapeDtypeStruct((N, 128), jnp.float32),
         jax.ShapeDtypeStruct((N, 128), jnp.float32),
         jax.ShapeDtypeStruct((N, 256), jnp.float32)],
    )(out1, sum1, sq1, gamma1.reshape(1, -1), beta1.reshape(1, -1), W_l2, W_r2)

    agg2a = seg128(p2a, src, dst, z128)
    agg2b = seg128(p2b, src, dst, z128)

    def tc1_l2(aa_ref, ab_ref, cinv_ref, r_ref, b_ref, out_ref, sum_ref, sq_ref):
        i = pl.program_id(0)
        cinv = cinv_ref[...]
        a = (aa_ref[0] + aa_ref[1]) * cinv
        b2 = (ab_ref[0] + ab_ref[1]) * cinv
        o = jnp.concatenate([a, b2], axis=1) + b_ref[...] + r_ref[...]
        out_ref[...] = o
        _stats_body_init(i, (sum_ref, sq_ref))
        sum_ref[...] += jnp.sum(o, axis=0, keepdims=True)
        sq_ref[...] += jnp.sum(o * o, axis=0, keepdims=True)

    out2, sum2, sq2 = _tc_call(
        tc1_l2, (R,),
        [_agg_spec(M, 128), _agg_spec(M, 128), _rows(M, 1), _rows(M, 256),
         _full((1, 256))],
        [_rows(M, 256), _full((1, 256)), _full((1, 256))],
        [jax.ShapeDtypeStruct((N, 256), jnp.float32),
         jax.ShapeDtypeStruct((1, 256), jnp.float32),
         jax.ShapeDtypeStruct((1, 256), jnp.float32)],
    )(agg2a, agg2b, cinv, r2, b_l2.reshape(1, -1))

    def tc2_l2(o_ref, sum_ref, sq_ref, g_ref, be_ref, wl_ref, wr_ref,
               p_ref, r_ref):
        h = _bn_relu(o_ref[...], sum_ref, sq_ref, g_ref, be_ref, float(N))
        p_ref[...] = jnp.dot(h, wl_ref[...], precision=HP, preferred_element_type=jnp.float32)
        r_ref[...] = jnp.dot(h, wr_ref[...], precision=HP, preferred_element_type=jnp.float32)

    p3, r3 = _tc_call(
        tc2_l2, (R,),
        [_rows(M, 256), _full((1, 256)), _full((1, 256)), _full((1, 256)),
         _full((1, 256)), _full((256, 128)), _full((256, 128))],
        [_rows(M, 128), _rows(M, 128)],
        [jax.ShapeDtypeStruct((N, 128), jnp.float32),
         jax.ShapeDtypeStruct((N, 128), jnp.float32)],
    )(out2, sum2, sq2, gamma2.reshape(1, -1), beta2.reshape(1, -1), W_l3, W_r3)

    agg3 = seg128(p3, src, dst, z128)

    def tc1_l3(a_ref, cinv_ref, r_ref, b_ref, out_ref, sum_ref, sq_ref):
        i = pl.program_id(0)
        o = (a_ref[0] + a_ref[1]) * cinv_ref[...] + b_ref[...] + r_ref[...]
        out_ref[...] = o
        _stats_body_init(i, (sum_ref, sq_ref))
        sum_ref[...] += jnp.sum(o, axis=0, keepdims=True)
        sq_ref[...] += jnp.sum(o * o, axis=0, keepdims=True)

    out3, sum3, sq3 = _tc_call(
        tc1_l3, (R,),
        [_agg_spec(M, 128), _rows(M, 1), _rows(M, 128), _full((1, 128))],
        [_rows(M, 128), _full((1, 128)), _full((1, 128))],
        [jax.ShapeDtypeStruct((N, 128), jnp.float32),
         jax.ShapeDtypeStruct((1, 128), jnp.float32),
         jax.ShapeDtypeStruct((1, 128), jnp.float32)],
    )(agg3, cinv, r3, b_l3.reshape(1, -1))

    def tc2_l3(o_ref, sum_ref, sq_ref, g_ref, be_ref, h_ref):
        h_ref[...] = _bn_relu(o_ref[...], sum_ref, sq_ref, g_ref, be_ref, float(N))

    h3 = _tc_call(
        tc2_l3, (R,),
        [_rows(M, 128), _full((1, 128)), _full((1, 128)), _full((1, 128)),
         _full((1, 128))],
        [_rows(M, 128)],
        [jax.ShapeDtypeStruct((N, 128), jnp.float32)],
    )(out3, sum3, sq3, gamma3.reshape(1, -1), beta3.reshape(1, -1))[0]

    return h3
